# baseline (speedup 1.0000x reference)
import functools

import jax
import jax.numpy as jnp
from jax import lax
from jax.experimental import pallas as pl
from jax.experimental.pallas import tpu as pltpu
from jax.experimental.pallas import tpu_sc as plsc

B = 1024
NC = 2
NS = 16
LANES = 16
NT = NC * NS
TC_BLK = 8192


def _round_up(x, m):
    return (x + m - 1) // m * m


def _make_sc_scatter(cnt):
    chunks = cnt // LANES
    mesh = plsc.VectorSubcoreMesh(core_axis_name="c", subcore_axis_name="s")

    @functools.partial(
        pl.kernel,
        mesh=mesh,
        compiler_params=pltpu.CompilerParams(
            use_tc_tiling_on_sc=False, needs_layout_passes=False),
        out_type=jax.ShapeDtypeStruct((NC, 4 * B), jnp.float32),
        scratch_types=[
            pltpu.VMEM((cnt + 2 * LANES,), jnp.int32),
            pltpu.VMEM((cnt,), jnp.float32),
            pltpu.VMEM((cnt,), jnp.float32),
            pltpu.VMEM((cnt,), jnp.float32),
            pltpu.VMEM((cnt,), jnp.float32),
            pltpu.VMEM((8 * LANES,), jnp.float32),
            pltpu.VMEM((4 * B,), jnp.float32),
            pltpu.VMEM((NS, 4 * B // NS), jnp.float32),
            pltpu.VMEM((4 * B // NS,), jnp.float32),
            pltpu.VMEM_SHARED((NS, 4 * B), jnp.float32),
            pltpu.SemaphoreType.DMA,
        ],
    )
    def sc_scatter(ids_hbm, q_hbm, x_hbm, y_hbm, z_hbm, out_hbm,
                   ids_v, v0, v1, v2, v3, ep8, acc, slab, res, shared, sem):
        c = lax.axis_index("c")
        s = lax.axis_index("s")
        wid = c * NS + s
        base = wid * cnt

        cps = [
            pltpu.async_copy(ids_hbm.at[pl.ds(base, cnt)],
                             ids_v.at[pl.ds(LANES, cnt)], sem),
            pltpu.async_copy(q_hbm.at[pl.ds(base, cnt)], v0, sem),
            pltpu.async_copy(x_hbm.at[pl.ds(base, cnt)], v1, sem),
            pltpu.async_copy(y_hbm.at[pl.ds(base, cnt)], v2, sem),
            pltpu.async_copy(z_hbm.at[pl.ds(base, cnt)], v3, sem),
        ]

        zeros16 = jnp.zeros((LANES,), jnp.float32)

        def zero_body(i, _):
            o = i * 8 * LANES
            for u in range(8):
                acc[pl.ds(o + u * LANES, LANES)] = zeros16
            return 0

        lax.fori_loop(0, 4 * B // (8 * LANES), zero_body, 0)
        for cp in cps:
            cp.wait()

        iot = lax.iota(jnp.int32, LANES)

        def chunk(o, epo):
            ids16 = ids_v[pl.ds(o + LANES, LANES)]
            prv = ids_v[pl.ds(o + LANES - 1, LANES)]
            nxt = ids_v[pl.ds(o + LANES + 1, LANES)]
            sm = (iot == 0) | (ids16 != prv)
            em = (iot == LANES - 1) | (ids16 != nxt)
            st = plsc.cummax(jnp.where(sm, iot, 0))
            idx4 = ids16 * 4
            for comp, vref in enumerate((v0, v1, v2, v3)):
                v = vref[pl.ds(o, LANES)]
                cs = plsc.cumsum(v)
                ep8[pl.ds(epo + comp * LANES, LANES)] = cs - v
                pb = plsc.load_gather(ep8, [epo + comp * LANES + st])
                plsc.addupdate_scatter(acc, [idx4 + comp], cs - pb, mask=em)

        def body(i, _):
            o = i * 2 * LANES
            chunk(o, 0)
            chunk(o + LANES, 4 * LANES)
            return 0

        lax.fori_loop(0, chunks // 2, body, 0)

        pltpu.sync_copy(acc, shared.at[s])
        plsc.subcore_barrier()
        cols = 4 * B // NS
        col0 = s * cols
        pltpu.sync_copy(shared.at[:, pl.ds(col0, cols)], slab)

        def red_body(j, _):
            off = j * LANES
            a = slab[0, pl.ds(off, LANES)]
            for k in range(1, NS):
                a = a + slab[k, pl.ds(off, LANES)]
            res[pl.ds(off, LANES)] = a
            return 0

        lax.fori_loop(0, cols // LANES, red_body, 0)
        pltpu.sync_copy(res, out_hbm.at[c, pl.ds(col0, cols)])

    return sc_scatter


def _tc_half_body(n_valid, lo, p1_ref, xyzt_ref, w_ref, b_ref,
                  oq, ox, oy, oz):
    i = pl.program_id(0)
    q = lax.dot_general(
        w_ref[...], p1_ref[...], (((1,), (1,)), ((), ())),
        preferred_element_type=jnp.float32,
    ) + b_ref[0, 0]
    col = lo + i * TC_BLK + lax.broadcasted_iota(jnp.int32, (1, TC_BLK), 1)
    valid = col < n_valid
    q = jnp.where(valid, q, 0.0)
    qd = jnp.where(valid, xyzt_ref[...] * q, 0.0)
    oq[...] = q[0]
    ox[...] = qd[0]
    oy[...] = qd[1]
    oz[...] = qd[2]


def kernel(atom_batch, p1, xyz, W, b):
    n, d = p1.shape
    cnt = _round_up(-(-n // NT), 2 * LANES)
    npad = NT * cnt
    nblk = -(-npad // TC_BLK)

    ids32 = atom_batch.astype(jnp.int32)
    ids_pad = jnp.concatenate(
        [ids32, jnp.full((npad - n,), B - 1, jnp.int32)])
    xyz_t = xyz.T
    b2 = b.reshape(1, 1)

    def tc_call(blk_off, nblk_h, out_len, body):
        vec = jax.ShapeDtypeStruct((out_len,), jnp.float32)
        return pl.pallas_call(
            body,
            grid=(nblk_h,),
            in_specs=[
                pl.BlockSpec((TC_BLK, d), lambda i: (i + blk_off, 0)),
                pl.BlockSpec((3, TC_BLK), lambda i: (0, i + blk_off)),
                pl.BlockSpec((1, d), lambda i: (0, 0)),
                pl.BlockSpec((1, 1), lambda i: (0, 0)),
            ],
            out_specs=[pl.BlockSpec((TC_BLK,), lambda i: (i,))] * 4,
            out_shape=[vec] * 4,
        )(p1, xyz_t, W, b2)

    nparts = 4
    per = nblk // nparts
    sizes = [per] * (nparts - 1) + [nblk - per * (nparts - 1)]
    partials = []
    off = 0
    for nb in sizes:
        lo = off * TC_BLK
        out_len = min(nb * TC_BLK, npad - lo)
        outs = tc_call(off, nb, out_len,
                       functools.partial(_tc_half_body, n, lo))
        partials.append(
            _make_sc_scatter(out_len // NT)(ids_pad[lo:lo + out_len], *outs))
        off += nb
    tot = functools.reduce(
        lambda a, x: a + x[0] + x[1], partials,
        jnp.zeros((4 * B,), jnp.float32)).reshape(B, 4)
    return tot[:, 0], tot[:, 1:4]


# speedup vs baseline: 1.0130x; 1.0130x over previous
import functools

import jax
import jax.numpy as jnp
from jax import lax
from jax.experimental import pallas as pl
from jax.experimental.pallas import tpu as pltpu
from jax.experimental.pallas import tpu_sc as plsc

B = 1024
NC = 2
NS = 16
LANES = 16
NT = NC * NS
TC_BLK = 8192


def _round_up(x, m):
    return (x + m - 1) // m * m


def _make_sc_scatter(cnt):
    chunks = cnt // LANES
    mesh = plsc.VectorSubcoreMesh(core_axis_name="c", subcore_axis_name="s")

    @functools.partial(
        pl.kernel,
        mesh=mesh,
        compiler_params=pltpu.CompilerParams(
            use_tc_tiling_on_sc=False, needs_layout_passes=False),
        out_type=jax.ShapeDtypeStruct((NC, 4 * B), jnp.float32),
        scratch_types=[
            pltpu.VMEM((cnt + 2 * LANES,), jnp.int32),
            pltpu.VMEM((cnt,), jnp.float32),
            pltpu.VMEM((cnt,), jnp.float32),
            pltpu.VMEM((cnt,), jnp.float32),
            pltpu.VMEM((cnt,), jnp.float32),
            pltpu.VMEM((8 * LANES,), jnp.float32),
            pltpu.VMEM((4 * B,), jnp.float32),
            pltpu.VMEM((NS, 4 * B // NS), jnp.float32),
            pltpu.VMEM((4 * B // NS,), jnp.float32),
            pltpu.VMEM_SHARED((NS, 4 * B), jnp.float32),
            pltpu.SemaphoreType.DMA,
        ],
    )
    def sc_scatter(ids_hbm, q_hbm, x_hbm, y_hbm, z_hbm, out_hbm,
                   ids_v, v0, v1, v2, v3, ep8, acc, slab, res, shared, sem):
        c = lax.axis_index("c")
        s = lax.axis_index("s")
        wid = c * NS + s
        base = wid * cnt

        cps = [
            pltpu.async_copy(ids_hbm.at[pl.ds(base, cnt)],
                             ids_v.at[pl.ds(LANES, cnt)], sem),
            pltpu.async_copy(q_hbm.at[pl.ds(base, cnt)], v0, sem),
            pltpu.async_copy(x_hbm.at[pl.ds(base, cnt)], v1, sem),
            pltpu.async_copy(y_hbm.at[pl.ds(base, cnt)], v2, sem),
            pltpu.async_copy(z_hbm.at[pl.ds(base, cnt)], v3, sem),
        ]

        zeros16 = jnp.zeros((LANES,), jnp.float32)

        def zero_body(i, _):
            o = i * 8 * LANES
            for u in range(8):
                acc[pl.ds(o + u * LANES, LANES)] = zeros16
            return 0

        lax.fori_loop(0, 4 * B // (8 * LANES), zero_body, 0)
        for cp in cps:
            cp.wait()

        iot = lax.iota(jnp.int32, LANES)

        def chunk(o, epo):
            ids16 = ids_v[pl.ds(o + LANES, LANES)]
            prv = ids_v[pl.ds(o + LANES - 1, LANES)]
            nxt = ids_v[pl.ds(o + LANES + 1, LANES)]
            sm = (iot == 0) | (ids16 != prv)
            em = (iot == LANES - 1) | (ids16 != nxt)
            st = plsc.cummax(jnp.where(sm, iot, 0))
            idx4 = ids16 * 4
            for comp, vref in enumerate((v0, v1, v2, v3)):
                v = vref[pl.ds(o, LANES)]
                cs = plsc.cumsum(v)
                ep8[pl.ds(epo + comp * LANES, LANES)] = cs - v
                pb = plsc.load_gather(ep8, [epo + comp * LANES + st])
                plsc.addupdate_scatter(acc, [idx4 + comp], cs - pb, mask=em)

        def body(i, _):
            o = i * 2 * LANES
            chunk(o, 0)
            chunk(o + LANES, 4 * LANES)
            return 0

        lax.fori_loop(0, chunks // 2, body, 0)

        pltpu.sync_copy(acc, shared.at[s])
        plsc.subcore_barrier()
        cols = 4 * B // NS
        col0 = s * cols
        pltpu.sync_copy(shared.at[:, pl.ds(col0, cols)], slab)

        def red_body(j, _):
            off = j * LANES
            a = slab[0, pl.ds(off, LANES)]
            for k in range(1, NS):
                a = a + slab[k, pl.ds(off, LANES)]
            res[pl.ds(off, LANES)] = a
            return 0

        lax.fori_loop(0, cols // LANES, red_body, 0)
        pltpu.sync_copy(res, out_hbm.at[c, pl.ds(col0, cols)])

    return sc_scatter


def _tc_half_body(n_valid, lo, p1_ref, xyzt_ref, w_ref, b_ref,
                  oq, ox, oy, oz):
    i = pl.program_id(0)
    q = lax.dot_general(
        w_ref[...], p1_ref[...], (((1,), (1,)), ((), ())),
        preferred_element_type=jnp.float32,
    ) + b_ref[0, 0]
    col = lo + i * TC_BLK + lax.broadcasted_iota(jnp.int32, (1, TC_BLK), 1)
    valid = col < n_valid
    q = jnp.where(valid, q, 0.0)
    qd = jnp.where(valid, xyzt_ref[...] * q, 0.0)
    oq[...] = q[0]
    ox[...] = qd[0]
    oy[...] = qd[1]
    oz[...] = qd[2]


def kernel(atom_batch, p1, xyz, W, b):
    n, d = p1.shape
    cnt = _round_up(-(-n // NT), 2 * LANES)
    npad = NT * cnt
    nblk = -(-npad // TC_BLK)

    ids32 = atom_batch.astype(jnp.int32)
    ids_pad = jnp.concatenate(
        [ids32, jnp.full((npad - n,), B - 1, jnp.int32)])
    xyz_t = xyz.T
    b2 = b.reshape(1, 1)

    def tc_call(blk_off, nblk_h, out_len, body):
        vec = jax.ShapeDtypeStruct((out_len,), jnp.float32)
        return pl.pallas_call(
            body,
            grid=(nblk_h,),
            in_specs=[
                pl.BlockSpec((TC_BLK, d), lambda i: (i + blk_off, 0)),
                pl.BlockSpec((3, TC_BLK), lambda i: (0, i + blk_off)),
                pl.BlockSpec((1, d), lambda i: (0, 0)),
                pl.BlockSpec((1, 1), lambda i: (0, 0)),
            ],
            out_specs=[pl.BlockSpec((TC_BLK,), lambda i: (i,))] * 4,
            out_shape=[vec] * 4,
        )(p1, xyz_t, W, b2)

    sizes = [(nblk * 3) // 5, nblk - (nblk * 3) // 5]
    partials = []
    off = 0
    for nb in sizes:
        lo = off * TC_BLK
        out_len = min(nb * TC_BLK, npad - lo)
        outs = tc_call(off, nb, out_len,
                       functools.partial(_tc_half_body, n, lo))
        partials.append(
            _make_sc_scatter(out_len // NT)(ids_pad[lo:lo + out_len], *outs))
        off += nb
    tot = functools.reduce(
        lambda a, x: a + x[0] + x[1], partials,
        jnp.zeros((4 * B,), jnp.float32)).reshape(B, 4)
    return tot[:, 0], tot[:, 1:4]
